# Initial kernel scaffold; baseline (speedup 1.0000x reference)
import jax
import jax.numpy as jnp
from jax.experimental import pallas as pl


def kernel(embeddings, labels, num_classes):
    raise NotImplementedError("write your pallas kernel here")


# speedup vs baseline: 5.6880x; 5.6880x over previous
import functools

import jax
import jax.numpy as jnp
from jax import lax
from jax.experimental import pallas as pl
from jax.experimental.pallas import tpu as pltpu
from jax.experimental.pallas import tpu_sc as plsc

N = 16384
D = 64
C = 100
C_PAD = 112
NC, NS = 2, 16
NW = NC * NS
ROWS_PER_W = N // NW
GROUPS = 4
GROUP = ROWS_PER_W // GROUPS
DELTA = 100.0


def _sc_segment_sums(emb4, lab3):
  mesh = plsc.VectorSubcoreMesh(
      core_axis_name="c", subcore_axis_name="s", num_cores=NC, num_subcores=NS
  )

  @functools.partial(
      pl.kernel,
      out_type=[
          jax.ShapeDtypeStruct((NC, C_PAD, D), jnp.float32),
          jax.ShapeDtypeStruct((NC, C_PAD, 16), jnp.float32),
      ],
      mesh=mesh,
      scratch_types=[
          pltpu.VMEM((GROUPS, GROUP, D), jnp.float32),
          pltpu.VMEM((GROUPS, GROUP), jnp.int32),
          pltpu.VMEM((GROUP, 16), jnp.float32),
          pltpu.VMEM((7, D), jnp.float32),
          pltpu.VMEM((7, 16), jnp.float32),
          pltpu.VMEM_SHARED((C_PAD, D), jnp.float32),
          pltpu.VMEM_SHARED((C_PAD, 16), jnp.float32),
          pltpu.SemaphoreType.DMA,
      ],
  )
  def seg_kernel(
      emb_hbm, lab_hbm, out_sum, out_cnt,
      rows_v, idx_v, ones_v, zs_v, zc_v, acc_sum, acc_cnt, sem,
  ):
    cid = lax.axis_index("c")
    sid = lax.axis_index("s")
    wid = cid * NS + sid

    rows_cp = pltpu.async_copy(emb_hbm.at[wid], rows_v, sem)
    pltpu.sync_copy(lab_hbm.at[wid], idx_v)

    one16 = jnp.ones((16,), jnp.float32)
    zero16 = jnp.zeros((16,), jnp.float32)

    @pl.loop(0, GROUP)
    def _(i):
      ones_v[i, :] = one16

    @pl.loop(0, 7)
    def _(r):
      zc_v[r, :] = zero16

      @pl.loop(0, D // 16)
      def _(j):
        zs_v[r, pl.ds(j * 16, 16)] = zero16

    pltpu.sync_copy(zs_v, acc_sum.at[pl.ds(sid * 7, 7)])
    pltpu.sync_copy(zc_v, acc_cnt.at[pl.ds(sid * 7, 7)])
    plsc.subcore_barrier()

    rows_cp.wait()
    for g in range(GROUPS):
      pltpu.sync_copy(rows_v.at[g], acc_sum.at[idx_v.at[g]], add=True)
      pltpu.sync_copy(ones_v, acc_cnt.at[idx_v.at[g]], add=True)
    plsc.subcore_barrier()

    @pl.when(sid == 0)
    def _():
      pltpu.sync_copy(acc_sum, out_sum.at[cid])
      pltpu.sync_copy(acc_cnt, out_cnt.at[cid])

  return seg_kernel(emb4, lab3)


def _tc_sumsq(emb):
  blocks = 8

  def body(x_ref, o_ref):
    @pl.when(pl.program_id(0) == 0)
    def _():
      o_ref[0, 0] = 0.0

    x = x_ref[...]
    o_ref[0, 0] += jnp.sum(x * x)

  return pl.pallas_call(
      body,
      grid=(blocks,),
      in_specs=[pl.BlockSpec((N // blocks, D), lambda i: (i, 0))],
      out_specs=pl.BlockSpec(memory_space=pltpu.SMEM),
      out_shape=jax.ShapeDtypeStruct((1, 1), jnp.float32),
  )(emb)


def _tc_finish(psum, pcnt, ssq, nc_arr):
  def body(ps_ref, pc_ref, ssq_ref, nc_ref, o_ref):
    sums = ps_ref[0] + ps_ref[1]
    cntf = pc_ref[0] + pc_ref[1]
    cnt = cntf[:, 0:1]
    safe = jnp.maximum(cnt, 1.0)
    mean = sums / safe
    wnorm = jnp.sum(sums * sums, axis=1, keepdims=True) / safe
    intra = ssq_ref[0, 0] - jnp.sum(wnorm)
    pm = mean[:, None, :] - mean[None, :, :]
    d2 = jnp.sum(pm * pm, axis=-1)
    ii = lax.broadcasted_iota(jnp.int32, (C_PAD, 1), 0)
    nonempty = (cnt > 0.0) & (ii < nc_ref[0, 0])
    ri = lax.broadcasted_iota(jnp.int32, (C_PAD, C_PAD), 0)
    ci = lax.broadcasted_iota(jnp.int32, (C_PAD, C_PAD), 1)
    pair_mask = nonempty & nonempty.reshape(1, C_PAD) & (ri != ci)
    inter = jnp.min(jnp.where(pair_mask, d2, jnp.inf))
    loss = intra / N - jnp.minimum(DELTA, inter)
    o_ref[0, 0] = loss

  return pl.pallas_call(
      body,
      in_specs=[
          pl.BlockSpec(memory_space=pltpu.VMEM),
          pl.BlockSpec(memory_space=pltpu.VMEM),
          pl.BlockSpec(memory_space=pltpu.SMEM),
          pl.BlockSpec(memory_space=pltpu.SMEM),
      ],
      out_specs=pl.BlockSpec(memory_space=pltpu.SMEM),
      out_shape=jax.ShapeDtypeStruct((1, 1), jnp.float32),
  )(psum, pcnt, ssq, nc_arr)


def kernel(embeddings, labels, num_classes):
  emb = embeddings.astype(jnp.float32)
  lab = labels.astype(jnp.int32)
  emb4 = emb.reshape(NW, GROUPS, GROUP, D)
  lab3 = lab.reshape(NW, GROUPS, GROUP)
  psum, pcnt = _sc_segment_sums(emb4, lab3)
  ssq = _tc_sumsq(emb)
  nc_arr = jnp.asarray(num_classes, jnp.int32).reshape(1, 1)
  loss = _tc_finish(psum, pcnt, ssq, nc_arr)
  return loss.reshape(1)


# speedup vs baseline: 5.8468x; 1.0279x over previous
import functools

import jax
import jax.numpy as jnp
from jax import lax
from jax.experimental import pallas as pl
from jax.experimental.pallas import tpu as pltpu
from jax.experimental.pallas import tpu_sc as plsc

N = 16384
D = 64
C = 100
C_PAD = 112
NC, NS = 2, 16
NW = NC * NS
ROWS_PER_W = N // NW
GROUPS = 4
GROUP = ROWS_PER_W // GROUPS
DELTA = 100.0


def _sc_segment_sums(emb4, lab3):
  mesh = plsc.VectorSubcoreMesh(
      core_axis_name="c", subcore_axis_name="s", num_cores=NC, num_subcores=NS
  )

  @functools.partial(
      pl.kernel,
      out_type=[
          jax.ShapeDtypeStruct((NC, C_PAD, D), jnp.float32),
          jax.ShapeDtypeStruct((NC, C_PAD, 16), jnp.float32),
          jax.ShapeDtypeStruct((NC, NS, 16), jnp.float32),
      ],
      mesh=mesh,
      scratch_types=[
          pltpu.VMEM((GROUPS, GROUP, D), jnp.float32),
          pltpu.VMEM((GROUPS, GROUP), jnp.int32),
          pltpu.VMEM((GROUP, 16), jnp.float32),
          pltpu.VMEM((7, D), jnp.float32),
          pltpu.VMEM((7, 16), jnp.float32),
          pltpu.VMEM((16,), jnp.float32),
          pltpu.VMEM_SHARED((C_PAD, D), jnp.float32),
          pltpu.VMEM_SHARED((C_PAD, 16), jnp.float32),
          pltpu.SemaphoreType.DMA,
      ],
  )
  def seg_kernel(
      emb_hbm, lab_hbm, out_sum, out_cnt, out_ssq,
      rows_v, idx_v, ones_v, zs_v, zc_v, ssq_v, acc_sum, acc_cnt, sem,
  ):
    cid = lax.axis_index("c")
    sid = lax.axis_index("s")
    wid = cid * NS + sid

    rows_cp = pltpu.async_copy(emb_hbm.at[wid], rows_v, sem)
    pltpu.sync_copy(lab_hbm.at[wid], idx_v)

    one16 = jnp.ones((16,), jnp.float32)
    zero16 = jnp.zeros((16,), jnp.float32)

    @pl.loop(0, GROUP)
    def _(i):
      ones_v[i, :] = one16

    @pl.loop(0, 7)
    def _(r):
      zc_v[r, :] = zero16

      @pl.loop(0, D // 16)
      def _(j):
        zs_v[r, pl.ds(j * 16, 16)] = zero16

    pltpu.sync_copy(zs_v, acc_sum.at[pl.ds(sid * 7, 7)])
    pltpu.sync_copy(zc_v, acc_cnt.at[pl.ds(sid * 7, 7)])
    plsc.subcore_barrier()

    rows_cp.wait()
    zacc = jnp.zeros((16,), jnp.float32)
    accs = (zacc, zacc, zacc, zacc)
    for g in range(GROUPS):

      @pl.loop(0, GROUP, init_carry=accs, unroll=4)
      def accs_loop(i, carry, g=g):
        new = []
        for j in range(D // 16):
          v = rows_v[g, i, pl.ds(j * 16, 16)]
          new.append(carry[j] + v * v)
        return tuple(new)

      accs = accs_loop

    ssq_v[...] = (accs[0] + accs[1]) + (accs[2] + accs[3])
    pltpu.sync_copy(ssq_v, out_ssq.at[cid, sid])
    for g in range(GROUPS):
      pltpu.sync_copy(rows_v.at[g], acc_sum.at[idx_v.at[g]], add=True)
      pltpu.sync_copy(ones_v, acc_cnt.at[idx_v.at[g]], add=True)
    plsc.subcore_barrier()

    @pl.when(sid == 0)
    def _():
      pltpu.sync_copy(acc_sum, out_sum.at[cid])
      pltpu.sync_copy(acc_cnt, out_cnt.at[cid])

  return seg_kernel(emb4, lab3)


def _tc_finish(psum, pcnt, ssq, nc_arr):
  def body(ps_ref, pc_ref, ssq_ref, nc_ref, o_ref):
    sums = ps_ref[0] + ps_ref[1]
    cntf = pc_ref[0] + pc_ref[1]
    cnt = cntf[:, 0:1]
    safe = jnp.maximum(cnt, 1.0)
    mean = sums / safe
    wnorm = jnp.sum(sums * sums, axis=1, keepdims=True) / safe
    ssq = jnp.sum(ssq_ref[0] + ssq_ref[1])
    intra = ssq - jnp.sum(wnorm)
    pm = mean[:, None, :] - mean[None, :, :]
    d2 = jnp.sum(pm * pm, axis=-1)
    ii = lax.broadcasted_iota(jnp.int32, (C_PAD, 1), 0)
    nonempty = (cnt > 0.0) & (ii < nc_ref[0, 0])
    ri = lax.broadcasted_iota(jnp.int32, (C_PAD, C_PAD), 0)
    ci = lax.broadcasted_iota(jnp.int32, (C_PAD, C_PAD), 1)
    pair_mask = nonempty & nonempty.reshape(1, C_PAD) & (ri != ci)
    inter = jnp.min(jnp.where(pair_mask, d2, jnp.inf))
    loss = intra / N - jnp.minimum(DELTA, inter)
    o_ref[0, 0] = loss

  return pl.pallas_call(
      body,
      in_specs=[
          pl.BlockSpec(memory_space=pltpu.VMEM),
          pl.BlockSpec(memory_space=pltpu.VMEM),
          pl.BlockSpec(memory_space=pltpu.VMEM),
          pl.BlockSpec(memory_space=pltpu.SMEM),
      ],
      out_specs=pl.BlockSpec(memory_space=pltpu.SMEM),
      out_shape=jax.ShapeDtypeStruct((1, 1), jnp.float32),
  )(psum, pcnt, ssq, nc_arr)


def kernel(embeddings, labels, num_classes):
  emb = embeddings.astype(jnp.float32)
  lab = labels.astype(jnp.int32)
  emb4 = emb.reshape(NW, GROUPS, GROUP, D)
  lab3 = lab.reshape(NW, GROUPS, GROUP)
  psum, pcnt, ssq = _sc_segment_sums(emb4, lab3)
  nc_arr = jnp.asarray(num_classes, jnp.int32).reshape(1, 1)
  loss = _tc_finish(psum, pcnt, ssq, nc_arr)
  return loss.reshape(1)


# speedup vs baseline: 6.3755x; 1.0904x over previous
import functools

import jax
import jax.numpy as jnp
from jax import lax
from jax.experimental import pallas as pl
from jax.experimental.pallas import tpu as pltpu
from jax.experimental.pallas import tpu_sc as plsc

N = 16384
D = 64
C = 100
C_PAD = 112
NC, NS = 2, 16
NW = NC * NS
ROWS_PER_W = N // NW
GROUPS = 4
GROUP = ROWS_PER_W // GROUPS
DELTA = 100.0


def _sc_segment_sums(emb, lab3):
  mesh = plsc.VectorSubcoreMesh(
      core_axis_name="c", subcore_axis_name="s", num_cores=NC, num_subcores=NS
  )

  @functools.partial(
      pl.kernel,
      out_type=[
          jax.ShapeDtypeStruct((NC, C_PAD, D), jnp.float32),
          jax.ShapeDtypeStruct((NC, NS, 16), jnp.float32),
      ],
      mesh=mesh,
      scratch_types=[
          pltpu.VMEM((ROWS_PER_W, D), jnp.float32),
          pltpu.VMEM((GROUPS, GROUP), jnp.int32),
          pltpu.VMEM((7, D), jnp.float32),
          pltpu.VMEM((16,), jnp.float32),
          pltpu.VMEM_SHARED((C_PAD, D), jnp.float32),
          pltpu.SemaphoreType.DMA,
      ],
  )
  def seg_kernel(
      emb_hbm, lab_hbm, out_sum, out_ssq,
      rows_v, idx_v, zs_v, ssq_v, acc_sum, sem,
  ):
    cid = lax.axis_index("c")
    sid = lax.axis_index("s")
    wid = cid * NS + sid

    rows_cp = pltpu.async_copy(
        emb_hbm.at[pl.ds(wid * ROWS_PER_W, ROWS_PER_W)], rows_v, sem
    )
    pltpu.sync_copy(lab_hbm.at[wid], idx_v)

    zero16 = jnp.zeros((16,), jnp.float32)

    @pl.loop(0, 7)
    def _(r):
      @pl.loop(0, D // 16)
      def _(j):
        zs_v[r, pl.ds(j * 16, 16)] = zero16

    pltpu.sync_copy(zs_v, acc_sum.at[pl.ds(sid * 7, 7)])
    plsc.subcore_barrier()

    rows_cp.wait()
    zacc = jnp.zeros((16,), jnp.float32)

    @pl.loop(0, ROWS_PER_W, init_carry=(zacc, zacc, zacc, zacc), unroll=4)
    def accs(i, carry):
      new = []
      for j in range(D // 16):
        v = rows_v[i, pl.ds(j * 16, 16)]
        new.append(carry[j] + v * v)
      return tuple(new)

    ssq_v[...] = (accs[0] + accs[1]) + (accs[2] + accs[3])
    pltpu.sync_copy(ssq_v, out_ssq.at[cid, sid])

    for g in range(GROUPS):
      pltpu.sync_copy(
          rows_v.at[pl.ds(g * GROUP, GROUP)],
          acc_sum.at[idx_v.at[g]],
          add=True,
      )
    plsc.subcore_barrier()

    @pl.when(sid == 0)
    def _():
      pltpu.sync_copy(acc_sum, out_sum.at[cid])

  return seg_kernel(emb, lab3)


def _tc_counts(lab2):
  def body(l_ref, o_ref):
    labs = l_ref[...]
    classes = lax.broadcasted_iota(jnp.int32, (C_PAD, 1, 1), 0)
    eq = (labs[None, :, :] == classes).astype(jnp.float32)
    o_ref[...] = jnp.sum(eq, axis=2)

  return pl.pallas_call(
      body,
      out_shape=jax.ShapeDtypeStruct((C_PAD, 128), jnp.float32),
  )(lab2)


def _tc_finish(psum, pcnt, ssq, nc_arr):
  def body(ps_ref, pc_ref, ssq_ref, nc_ref, o_ref):
    sums = ps_ref[0] + ps_ref[1]
    cnt = jnp.sum(pc_ref[...], axis=1, keepdims=True)
    safe = jnp.maximum(cnt, 1.0)
    mean = sums / safe
    wnorm = jnp.sum(sums * sums, axis=1, keepdims=True) / safe
    ssq = jnp.sum(ssq_ref[0] + ssq_ref[1])
    intra = ssq - jnp.sum(wnorm)
    gram = lax.dot_general(
        mean, mean, (((1,), (1,)), ((), ())),
        preferred_element_type=jnp.float32,
        precision=lax.Precision.HIGHEST,
    )
    n2 = jnp.sum(mean * mean, axis=1, keepdims=True)
    d2 = n2 + n2.reshape(1, C_PAD) - 2.0 * gram
    ii = lax.broadcasted_iota(jnp.int32, (C_PAD, 1), 0)
    nonempty = (cnt > 0.0) & (ii < nc_ref[0, 0])
    ri = lax.broadcasted_iota(jnp.int32, (C_PAD, C_PAD), 0)
    ci = lax.broadcasted_iota(jnp.int32, (C_PAD, C_PAD), 1)
    pair_mask = nonempty & nonempty.reshape(1, C_PAD) & (ri != ci)
    inter = jnp.min(jnp.where(pair_mask, d2, jnp.inf))
    loss = intra / N - jnp.minimum(DELTA, inter)
    o_ref[0, 0] = loss

  return pl.pallas_call(
      body,
      in_specs=[
          pl.BlockSpec(memory_space=pltpu.VMEM),
          pl.BlockSpec(memory_space=pltpu.VMEM),
          pl.BlockSpec(memory_space=pltpu.VMEM),
          pl.BlockSpec(memory_space=pltpu.SMEM),
      ],
      out_specs=pl.BlockSpec(memory_space=pltpu.SMEM),
      out_shape=jax.ShapeDtypeStruct((1, 1), jnp.float32),
  )(psum, pcnt, ssq, nc_arr)


def kernel(embeddings, labels, num_classes):
  emb = embeddings.astype(jnp.float32)
  lab = labels.astype(jnp.int32)
  lab3 = lab.reshape(NW, GROUPS, GROUP)
  psum, ssq = _sc_segment_sums(emb, lab3)
  pcnt = _tc_counts(lab.reshape(128, 128))
  nc_arr = jnp.asarray(num_classes, jnp.int32).reshape(1, 1)
  loss = _tc_finish(psum, pcnt, ssq, nc_arr)
  return loss.reshape(1)
